# baseline (speedup 1.0000x reference)
import jax
import jax.numpy as jnp
from jax.experimental import pallas as pl

N = 50000
V = 128
G = 256
EPS = 1e-05
BLK = 1000
NB = N // BLK


def _seg_kernel(pos_ref, batch_ref, seg_ref, cnt_ref):
    i = pl.program_id(0)
    x = pos_ref[...]
    nrm = jnp.sqrt(x[:, 0, :] * x[:, 0, :] + x[:, 1, :] * x[:, 1, :]
                   + x[:, 2, :] * x[:, 2, :])
    b = batch_ref[0, 0, :]
    oh = (jax.lax.broadcasted_iota(jnp.int32, (G, BLK), 0)
          == b[None, :]).astype(jnp.float32)
    part = jnp.dot(oh, nrm, preferred_element_type=jnp.float32)
    pcnt = jnp.sum(oh, axis=1)[None, :]

    @pl.when(i == 0)
    def _():
        seg_ref[...] = jnp.zeros_like(seg_ref)
        cnt_ref[...] = jnp.zeros_like(cnt_ref)

    seg_ref[...] += part
    cnt_ref[...] += pcnt


def _norm_kernel(pos_ref, batch_ref, seg_ref, cnt_ref, w_ref, out_ref):
    x = pos_ref[...]
    b = batch_ref[0, 0, :]
    cnt = jnp.maximum(cnt_ref[0, :], 1.0)
    mean = seg_ref[...] / cnt[:, None]
    oh = (b[:, None] == jax.lax.broadcasted_iota(jnp.int32, (BLK, G), 1)
          ).astype(jnp.float32)
    gm = jnp.dot(oh, mean, preferred_element_type=jnp.float32)
    denom = gm[:, None, :] + EPS
    w = w_ref[0, 0, :]
    out_ref[...] = x * w[None, None, :] / denom


def kernel(pos, weight, batch):
    b3 = batch.astype(jnp.int32).reshape(NB, 1, BLK)

    seg, cnt = pl.pallas_call(
        _seg_kernel,
        grid=(NB,),
        in_specs=[
            pl.BlockSpec((BLK, 3, V), lambda i: (i, 0, 0)),
            pl.BlockSpec((1, 1, BLK), lambda i: (i, 0, 0)),
        ],
        out_specs=[
            pl.BlockSpec((G, V), lambda i: (0, 0)),
            pl.BlockSpec((1, G), lambda i: (0, 0)),
        ],
        out_shape=[
            jax.ShapeDtypeStruct((G, V), jnp.float32),
            jax.ShapeDtypeStruct((1, G), jnp.float32),
        ],
    )(pos, b3)

    out = pl.pallas_call(
        _norm_kernel,
        grid=(NB,),
        in_specs=[
            pl.BlockSpec((BLK, 3, V), lambda i: (i, 0, 0)),
            pl.BlockSpec((1, 1, BLK), lambda i: (i, 0, 0)),
            pl.BlockSpec((G, V), lambda i: (0, 0)),
            pl.BlockSpec((1, G), lambda i: (0, 0)),
            pl.BlockSpec((1, 1, V), lambda i: (0, 0, 0)),
        ],
        out_specs=pl.BlockSpec((BLK, 3, V), lambda i: (i, 0, 0)),
        out_shape=jax.ShapeDtypeStruct((N, 3, V), jnp.float32),
    )(pos, b3, seg, cnt, weight)

    return out


# speedup vs baseline: 1.0162x; 1.0162x over previous
import jax
import jax.numpy as jnp
from jax.experimental import pallas as pl

N = 50000
V = 128
G = 256
EPS = 1e-05
BLK = 1000
NB = N // BLK


def _seg_kernel(pos_ref, batch_ref, seg_ref, cnt_ref):
    i = pl.program_id(0)
    x = pos_ref[...]
    nrm = jnp.sqrt(x[:, 0, :] * x[:, 0, :] + x[:, 1, :] * x[:, 1, :]
                   + x[:, 2, :] * x[:, 2, :])
    b = batch_ref[0, 0, :]
    oh = (jax.lax.broadcasted_iota(jnp.int32, (G, BLK), 0)
          == b[None, :]).astype(jnp.float32)
    part = jnp.dot(oh, nrm, preferred_element_type=jnp.float32)
    pcnt = jnp.sum(oh, axis=1)[None, :]

    @pl.when(i == 0)
    def _():
        seg_ref[...] = jnp.zeros_like(seg_ref)
        cnt_ref[...] = jnp.zeros_like(cnt_ref)

    seg_ref[...] += part
    cnt_ref[...] += pcnt


def _norm_kernel(pos_ref, batch_ref, seg_ref, cnt_ref, w_ref, out_ref):
    x = pos_ref[...]
    b = batch_ref[0, 0, :]
    cnt = jnp.maximum(cnt_ref[0, :], 1.0)
    mean = seg_ref[...] / cnt[:, None]
    oh = (b[:, None] == jax.lax.broadcasted_iota(jnp.int32, (BLK, G), 1)
          ).astype(jnp.float32)
    gm = jnp.dot(oh, mean, preferred_element_type=jnp.float32)
    w = w_ref[0, 0, :]
    scale = w[None, :] / (gm + EPS)
    out_ref[...] = x * scale[:, None, :]


def kernel(pos, weight, batch):
    b3 = batch.astype(jnp.int32).reshape(NB, 1, BLK)

    seg, cnt = pl.pallas_call(
        _seg_kernel,
        grid=(NB,),
        in_specs=[
            pl.BlockSpec((BLK, 3, V), lambda i: (i, 0, 0)),
            pl.BlockSpec((1, 1, BLK), lambda i: (i, 0, 0)),
        ],
        out_specs=[
            pl.BlockSpec((G, V), lambda i: (0, 0)),
            pl.BlockSpec((1, G), lambda i: (0, 0)),
        ],
        out_shape=[
            jax.ShapeDtypeStruct((G, V), jnp.float32),
            jax.ShapeDtypeStruct((1, G), jnp.float32),
        ],
    )(pos, b3)

    out = pl.pallas_call(
        _norm_kernel,
        grid=(NB,),
        in_specs=[
            pl.BlockSpec((BLK, 3, V), lambda i: (i, 0, 0)),
            pl.BlockSpec((1, 1, BLK), lambda i: (i, 0, 0)),
            pl.BlockSpec((G, V), lambda i: (0, 0)),
            pl.BlockSpec((1, G), lambda i: (0, 0)),
            pl.BlockSpec((1, 1, V), lambda i: (0, 0, 0)),
        ],
        out_specs=pl.BlockSpec((BLK, 3, V), lambda i: (i, 0, 0)),
        out_shape=jax.ShapeDtypeStruct((N, 3, V), jnp.float32),
    )(pos, b3, seg, cnt, weight)

    return out
